# baseline (speedup 1.0000x reference)
import functools

import jax
import jax.numpy as jnp
from jax.experimental import pallas as pl
from jax.experimental.pallas import tpu as pltpu

IN_F = 10
TILE_B = 8192


def _mlp_kernel(xa_ref, xb_ref, w1_ref, b1_ref, w2_ref, b2_ref, o_any,
                ybuf, osem, *, q, steps_per_core):
    i = pl.program_id(0)
    core = i // steps_per_core
    local = i - core * steps_per_core
    slot = jax.lax.rem(local, 2)

    w1 = w1_ref[...]
    w2 = w2_ref[...]
    b1 = b1_ref[...]
    b2 = b2_ref[...]

    def out_copy(sl, half, step):
        return pltpu.make_async_copy(
            ybuf.at[sl, half],
            o_any.at[half, pl.ds(step * TILE_B, TILE_B), :],
            osem.at[sl, half],
        )

    @pl.when(local >= 2)
    def _():
        out_copy(slot, 0, i).wait()
        out_copy(slot, 1, i).wait()

    h = jnp.dot(xa_ref[...], w1, preferred_element_type=jnp.float32) + b1
    h = jnp.maximum(h, 0.0)
    ybuf[slot, 0] = jnp.dot(h, w2, preferred_element_type=jnp.float32) + b2

    h = jnp.dot(xb_ref[...], w1, preferred_element_type=jnp.float32) + b1
    h = jnp.maximum(h, 0.0)
    ybuf[slot, 1] = jnp.dot(h, w2, preferred_element_type=jnp.float32) + b2

    out_copy(slot, 0, i).start()
    out_copy(slot, 1, i).start()

    @pl.when(local == steps_per_core - 1)
    def _():
        other = 1 - slot

        @pl.when(steps_per_core >= 2)
        def _():
            out_copy(other, 0, i).wait()
            out_copy(other, 1, i).wait()

        out_copy(slot, 0, i).wait()
        out_copy(slot, 1, i).wait()


def kernel(x, w1_t, b1_2d, w2_t, b2_2d):
    B = x.shape[0]
    q = B // (2 * TILE_B)
    steps_per_core = q // 2
    body = functools.partial(_mlp_kernel, q=q, steps_per_core=steps_per_core)
    vmem = pltpu.MemorySpace.VMEM
    y3 = pl.pallas_call(
        body,
        out_shape=jax.ShapeDtypeStruct((2, B // 2, IN_F), x.dtype),
        grid=(q,),
        in_specs=[
            pl.BlockSpec((TILE_B, IN_F), lambda i: (i, 0)),
            pl.BlockSpec((TILE_B, IN_F), lambda i, q=q: (i + q, 0)),
            pl.BlockSpec((IN_F, IN_F), lambda i: (0, 0), memory_space=vmem),
            pl.BlockSpec((1, IN_F), lambda i: (0, 0), memory_space=vmem),
            pl.BlockSpec((IN_F, IN_F), lambda i: (0, 0), memory_space=vmem),
            pl.BlockSpec((1, IN_F), lambda i: (0, 0), memory_space=vmem),
        ],
        out_specs=pl.BlockSpec(memory_space=pl.ANY),
        scratch_shapes=[
            pltpu.VMEM((2, 2, TILE_B, IN_F), jnp.float32),
            pltpu.SemaphoreType.DMA((2, 2)),
        ],
        compiler_params=pltpu.CompilerParams(
            dimension_semantics=("parallel",),
            vmem_limit_bytes=64 * 1024 * 1024,
        ),
        cost_estimate=pl.CostEstimate(
            flops=4 * B * IN_F * IN_F,
            transcendentals=0,
            bytes_accessed=2 * B * IN_F * 4,
        ),
    )(x, x, w1_t, b1_2d, w2_t, b2_2d)
    return jnp.reshape(y3, (B, IN_F))
